# Initial kernel scaffold; baseline (speedup 1.0000x reference)
import jax
import jax.numpy as jnp
from jax.experimental import pallas as pl


def kernel(bary_coords, pix_to_face, face_textures):
    raise NotImplementedError("write your pallas kernel here")


# speedup vs baseline: 54.5169x; 54.5169x over previous
import functools

import jax
import jax.numpy as jnp
from jax import lax
from jax.experimental import pallas as pl
from jax.experimental.pallas import tpu as pltpu
from jax.experimental.pallas import tpu_sc as plsc

N, H, W, K, F, C = 4, 512, 512, 8, 100000, 16
B = N * H * W
NW = 32
PXW = B // NW
P = 1024
NCH = PXW // P
GSZ = 128
NG = P // GSZ


def _tex_kernel(idx_hbm, bary_hbm, table_hbm, out_hbm,
                idx_v, bary_v, rows_v, out_v, sem):
    wid = lax.axis_index("s") * 2 + lax.axis_index("c")

    def chunk_body(ci, carry):
        base = pl.multiple_of(wid * PXW + ci * P, P)
        row0 = pl.multiple_of(base // GSZ, NG)

        pltpu.sync_copy(idx_hbm.at[pl.ds(row0, NG)], idx_v)
        pltpu.sync_copy(bary_hbm.at[pl.ds(3 * base, 3 * P)], bary_v)

        copies = [
            pltpu.async_copy(table_hbm.at[idx_v.at[j]], rows_v.at[j], sem)
            for j in range(NG)
        ]
        for cp in copies:
            cp.wait()

        def px_body(i, jj):
            p16 = (jj * GSZ + i) * 16
            b3 = (jj * GSZ + i) * 3
            r0 = rows_v[jj, i, pl.ds(0, 16)]
            r1 = rows_v[jj, i, pl.ds(16, 16)]
            r2 = rows_v[jj, i, pl.ds(32, 16)]
            b0 = plsc.load_gather(bary_v, [jnp.full((16,), b3, jnp.int32)])
            b1 = plsc.load_gather(bary_v, [jnp.full((16,), b3 + 1, jnp.int32)])
            b2 = plsc.load_gather(bary_v, [jnp.full((16,), b3 + 2, jnp.int32)])
            out_v[pl.ds(p16, 16)] = b0 * r0 + b1 * r1 + b2 * r2
            return jj

        for j in range(NG):
            lax.fori_loop(0, GSZ, px_body, j)

        pltpu.sync_copy(out_v, out_hbm.at[pl.ds(base * 16, P * 16)])
        return carry

    lax.fori_loop(0, NCH, chunk_body, 0)


@functools.partial(jax.jit, static_argnums=())
def _run(idx, bary, table):
    mesh = plsc.VectorSubcoreMesh(core_axis_name="c", subcore_axis_name="s")
    f = functools.partial(
        pl.kernel,
        mesh=mesh,
        compiler_params=pltpu.CompilerParams(
            needs_layout_passes=False, use_tc_tiling_on_sc=False),
        out_type=jax.ShapeDtypeStruct((B * 16,), jnp.float32),
        scratch_types=[
            pltpu.VMEM((NG, GSZ), jnp.int32),
            pltpu.VMEM((3 * P,), jnp.float32),
            pltpu.VMEM((NG, GSZ, 3 * C), jnp.float32),
            pltpu.VMEM((P * 16,), jnp.float32),
            pltpu.SemaphoreType.DMA,
        ],
    )(_tex_kernel)
    return f(idx, bary, table)


def kernel(bary_coords, pix_to_face, face_textures):
    idx = pix_to_face[:, :, :, 0].astype(jnp.int32).reshape(B // GSZ, GSZ)
    bary = bary_coords[:, :, :, 0, :].reshape(3 * B)
    table = face_textures.reshape(F, 3 * C)
    out = _run(idx, bary, table)
    return out.reshape(N, H, W, C)


# speedup vs baseline: 130.6556x; 2.3966x over previous
import functools

import jax
import jax.numpy as jnp
from jax import lax
from jax.experimental import pallas as pl
from jax.experimental.pallas import tpu as pltpu
from jax.experimental.pallas import tpu_sc as plsc

N, H, W, K, F, C = 4, 512, 512, 8, 100000, 16
B = N * H * W
NW = 32
PXW = B // NW
P = 1024
NCH = PXW // P
GSZ = 128
NG = P // GSZ


def _tex_kernel(pix_hbm, bary_hbm, table_hbm, out_hbm,
                idx_v, bary_v, rows_v, out_v, sem):
    wid = lax.axis_index("s") * 2 + lax.axis_index("c")

    def chunk_body(ci, carry):
        base = pl.multiple_of(wid * PXW + ci * P, P)
        row0 = pl.multiple_of(base // GSZ, NG)

        pltpu.sync_copy(pix_hbm.at[pl.ds(row0, NG)], idx_v)
        pltpu.sync_copy(bary_hbm.at[pl.ds(base, P), pl.ds(0, 3)], bary_v)

        copies = [
            pltpu.async_copy(table_hbm.at[idx_v.at[j]], rows_v.at[j], sem)
            for j in range(NG)
        ]
        for cp in copies:
            cp.wait()

        def px_body(i, jj):
            p = jj * GSZ + i
            r0 = rows_v[jj, i, pl.ds(0, 16)]
            r1 = rows_v[jj, i, pl.ds(16, 16)]
            r2 = rows_v[jj, i, pl.ds(32, 16)]
            pv = jnp.full((16,), p, jnp.int32)
            b0 = plsc.load_gather(bary_v, [pv, jnp.zeros((16,), jnp.int32)])
            b1 = plsc.load_gather(bary_v, [pv, jnp.ones((16,), jnp.int32)])
            b2 = plsc.load_gather(bary_v, [pv, jnp.full((16,), 2, jnp.int32)])
            out_v[pl.ds(p * 16, 16)] = b0 * r0 + b1 * r1 + b2 * r2
            return jj

        for j in range(NG):
            lax.fori_loop(0, GSZ, px_body, j)

        pltpu.sync_copy(out_v, out_hbm.at[pl.ds(base * 16, P * 16)])
        return carry

    lax.fori_loop(0, NCH, chunk_body, 0)


@jax.jit
def _run(pix, bary, table):
    mesh = plsc.VectorSubcoreMesh(core_axis_name="c", subcore_axis_name="s")
    f = functools.partial(
        pl.kernel,
        mesh=mesh,
        compiler_params=pltpu.CompilerParams(
            needs_layout_passes=False, use_tc_tiling_on_sc=False),
        out_type=jax.ShapeDtypeStruct((B * 16,), jnp.float32),
        scratch_types=[
            pltpu.VMEM((NG, GSZ), jnp.int32),
            pltpu.VMEM((P, 3), jnp.float32),
            pltpu.VMEM((NG, GSZ, 3 * C), jnp.float32),
            pltpu.VMEM((P * 16,), jnp.float32),
            pltpu.SemaphoreType.DMA,
        ],
    )(_tex_kernel)
    return f(pix, bary, table)


def kernel(bary_coords, pix_to_face, face_textures):
    pix = pix_to_face[:, :, :, 0].astype(jnp.int32).reshape(B // GSZ, GSZ)
    bary = bary_coords.reshape(B, K * 3)
    table = face_textures.reshape(F, 3 * C)
    out = _run(pix, bary, table)
    return out.reshape(N, H, W, C)
